# baseline (speedup 1.0000x reference)
import functools

import jax
import jax.numpy as jnp
from jax.experimental import pallas as pl
from jax.experimental.pallas import tpu as pltpu

_BM = 400
_NS = 5
_BH = _BM // _NS


def _body(adj0_ref, adj1_ref, adj2_ref, adj3_ref, adj4_ref,
          xk_ref, w1a_ref, w1b_ref, b1_ref,
          w2_ref, b2_ref, pa_ref, w3_ref, b3_ref, out_ref):
    i = pl.program_id(0)
    aggs = [jnp.dot(a[...], xk_ref[...], preferred_element_type=jnp.float32)
            for a in (adj0_ref, adj1_ref, adj2_ref, adj3_ref, adj4_ref)]
    agg = jnp.concatenate(aggs, axis=0)
    xr = xk_ref[pl.ds(i * _BM, _BM), :]
    h = jnp.dot(xr, w1a_ref[...], preferred_element_type=jnp.float32)
    h += jnp.dot(agg, w1b_ref[...], preferred_element_type=jnp.float32)
    h = jnp.maximum(h + b1_ref[...], 0.0)
    z = jnp.dot(h, w2_ref[...], preferred_element_type=jnp.float32) + b2_ref[...]
    p = jnp.where(z >= 0, z, pa_ref[...] * z)
    out_ref[...] = jnp.dot(p, w3_ref[...],
                           preferred_element_type=jnp.float32) + b3_ref[...]


@jax.jit
def kernel(x, adj, W1, b1, W2, b2, prelu_a, W3, b3):
    n, d = x.shape
    nhid = W2.shape[0]
    nclass = W3.shape[1]

    x_res = x.astype(jnp.bfloat16)
    w1a = W1[:d]
    w1b = W1[d:]
    b1r = b1.reshape(1, nhid)
    b2r = b2.reshape(1, nhid)
    par = prelu_a.reshape(1, nhid)
    b3r = b3.reshape(1, nclass)

    grid = (n // _BM,)
    out = pl.pallas_call(
        _body,
        grid=grid,
        in_specs=[
            *[pl.BlockSpec((_BH, n), functools.partial(
                lambda s, i: (_NS * i + s, 0), s))
              for s in range(_NS)],
            pl.BlockSpec((n, d), lambda i: (0, 0)),
            pl.BlockSpec((d, nhid), lambda i: (0, 0)),
            pl.BlockSpec((d, nhid), lambda i: (0, 0)),
            pl.BlockSpec((1, nhid), lambda i: (0, 0)),
            pl.BlockSpec((nhid, nhid), lambda i: (0, 0)),
            pl.BlockSpec((1, nhid), lambda i: (0, 0)),
            pl.BlockSpec((1, nhid), lambda i: (0, 0)),
            pl.BlockSpec((nhid, nclass), lambda i: (0, 0)),
            pl.BlockSpec((1, nclass), lambda i: (0, 0)),
        ],
        out_specs=pl.BlockSpec((_BM, nclass), lambda i: (i, 0)),
        out_shape=jax.ShapeDtypeStruct((n, nclass), jnp.float32),
        compiler_params=pltpu.CompilerParams(
            dimension_semantics=("parallel",)),
    )(*([adj] * _NS), x_res, w1a, w1b, b1r, W2, b2r, par, W3, b3r)
    return out


# speedup vs baseline: 1.7475x; 1.7475x over previous
import functools

import jax
import jax.numpy as jnp
from jax.experimental import pallas as pl
from jax.experimental.pallas import tpu as pltpu

_BM = 400
_NS = 2
_BH = _BM // _NS


def _body(adj0_ref, adj1_ref,
          xk_ref, w1a_ref, w1b_ref, b1_ref,
          w2_ref, b2_ref, pa_ref, w3_ref, b3_ref, out_ref):
    i = pl.program_id(0)
    aggs = [jnp.dot(a[...], xk_ref[...], preferred_element_type=jnp.float32)
            for a in (adj0_ref, adj1_ref)]
    agg = jnp.concatenate(aggs, axis=0)
    xr = xk_ref[pl.ds(i * _BM, _BM), :]
    h = jnp.dot(xr, w1a_ref[...], preferred_element_type=jnp.float32)
    h += jnp.dot(agg, w1b_ref[...], preferred_element_type=jnp.float32)
    h = jnp.maximum(h + b1_ref[...], 0.0)
    z = jnp.dot(h, w2_ref[...], preferred_element_type=jnp.float32) + b2_ref[...]
    p = jnp.where(z >= 0, z, pa_ref[...] * z)
    out_ref[...] = jnp.dot(p, w3_ref[...],
                           preferred_element_type=jnp.float32) + b3_ref[...]


@jax.jit
def kernel(x, adj, W1, b1, W2, b2, prelu_a, W3, b3):
    n, d = x.shape
    nhid = W2.shape[0]
    nclass = W3.shape[1]

    x_res = x
    w1a = W1[:d]
    w1b = W1[d:]
    b1r = b1.reshape(1, nhid)
    b2r = b2.reshape(1, nhid)
    par = prelu_a.reshape(1, nhid)
    b3r = b3.reshape(1, nclass)

    grid = (n // _BM,)
    out = pl.pallas_call(
        _body,
        grid=grid,
        in_specs=[
            *[pl.BlockSpec((_BH, n), functools.partial(
                lambda s, i: (_NS * i + s, 0), s))
              for s in range(_NS)],
            pl.BlockSpec((n, d), lambda i: (0, 0)),
            pl.BlockSpec((d, nhid), lambda i: (0, 0)),
            pl.BlockSpec((d, nhid), lambda i: (0, 0)),
            pl.BlockSpec((1, nhid), lambda i: (0, 0)),
            pl.BlockSpec((nhid, nhid), lambda i: (0, 0)),
            pl.BlockSpec((1, nhid), lambda i: (0, 0)),
            pl.BlockSpec((1, nhid), lambda i: (0, 0)),
            pl.BlockSpec((nhid, nclass), lambda i: (0, 0)),
            pl.BlockSpec((1, nclass), lambda i: (0, 0)),
        ],
        out_specs=pl.BlockSpec((_BM, nclass), lambda i: (i, 0)),
        out_shape=jax.ShapeDtypeStruct((n, nclass), jnp.float32),
        compiler_params=pltpu.CompilerParams(
            dimension_semantics=("parallel",),
            vmem_limit_bytes=128 * 1024 * 1024),
    )(*([adj] * _NS), x_res, w1a, w1b, b1r, W2, b2r, par, W3, b3r)
    return out
